# baseline (speedup 1.0000x reference)
import functools
import jax
import jax.numpy as jnp
from jax import lax
from jax.experimental import pallas as pl
from jax.experimental.pallas import tpu as pltpu
from jax.experimental.pallas import tpu_sc as plsc

D = 1024
V = 100000
NW = 32
CW = 3200
R = 16
NG = D // R
NJ = CW // 16


def _sc_body(x_hbm, w_hbm, b_hbm, vals_hbm, idx_hbm,
             xv, acc, wb0, wb1, st_v, st_i, sem0, sem1):
    cid = lax.axis_index("c")
    sid = lax.axis_index("s")
    wid = sid * 2 + cid
    lo = ((wid * (V - CW)) // (NW - 1)) // 8 * 8

    pltpu.sync_copy(x_hbm, xv)
    pltpu.sync_copy(b_hbm.at[pl.ds(lo, CW)], acc)

    def start(g, buf, sem):
        pltpu.async_copy(
            w_hbm.at[pl.ds(g * R, R), pl.ds(lo, CW)], buf, sem)

    def accumulate(g, buf):
        xg = xv[pl.ds(g * R, 16)]
        xs = [xg[r] for r in range(R)]

        @plsc.parallel_loop(0, NJ, unroll=4)
        def jbody(j):
            s = pl.ds(j * 16, 16)
            parts = []
            for c in range(4):
                p = xs[4 * c] * buf[4 * c, s]
                for r in range(4 * c + 1, 4 * c + 4):
                    p = p + xs[r] * buf[r, s]
                parts.append(p)
            plsc.addupdate(acc.at[s], (parts[0] + parts[1]) +
                           (parts[2] + parts[3]))

    start(0, wb0, sem0)
    start(1, wb1, sem1)

    def gbody(t, _):
        for bi, (buf, sem) in enumerate(((wb0, sem0), (wb1, sem1))):
            g = 2 * t + bi
            pltpu.make_async_copy(
                w_hbm.at[pl.ds(g * R, R), pl.ds(lo, CW)], buf, sem).wait()

            accumulate(g, buf)

            @pl.when(g + 2 < NG)
            def _():
                start(g + 2, buf, sem)
        return 0

    lax.fori_loop(0, NG // 2, gbody, 0)

    def rbody(j, carry):
        vm, vi = carry
        v = acc[pl.ds(j * 16, 16)]
        col = lo + j * 16 + lax.iota(jnp.int32, 16)
        upd = v > vm
        return jnp.where(upd, v, vm), jnp.where(upd, col, vi)

    vm0 = acc[pl.ds(0, 16)]
    vi0 = lo + lax.iota(jnp.int32, 16)
    vm, vi = lax.fori_loop(1, NJ, rbody, (vm0, vi0))

    st_v[...] = vm
    st_i[...] = vi
    pltpu.sync_copy(st_v, vals_hbm.at[wid])
    pltpu.sync_copy(st_i, idx_hbm.at[wid])


_sc_top1 = functools.partial(
    pl.kernel,
    out_type=[
        jax.ShapeDtypeStruct((NW, 16), jnp.float32),
        jax.ShapeDtypeStruct((NW, 16), jnp.int32),
    ],
    mesh=plsc.VectorSubcoreMesh(core_axis_name="c", subcore_axis_name="s"),
    compiler_params=pltpu.CompilerParams(use_tc_tiling_on_sc=False),
    scratch_types=[
        pltpu.VMEM((D,), jnp.float32),
        pltpu.VMEM((CW,), jnp.float32),
        pltpu.VMEM((R, CW), jnp.float32),
        pltpu.VMEM((R, CW), jnp.float32),
        pltpu.VMEM((16,), jnp.float32),
        pltpu.VMEM((16,), jnp.int32),
        pltpu.SemaphoreType.DMA,
        pltpu.SemaphoreType.DMA,
    ],
)(_sc_body)


def _merge_body(vals_ref, idx_ref, out_ref):
    m = jnp.max(vals_ref[...])
    out_ref[0] = jnp.min(jnp.where(vals_ref[...] == m, idx_ref[...], V))


def kernel(x, W, b):
    vals, idx = _sc_top1(x[0], W, b)
    topk_id = pl.pallas_call(
        _merge_body,
        out_specs=pl.BlockSpec(memory_space=pltpu.SMEM),
        out_shape=jax.ShapeDtypeStruct((1,), jnp.int32),
    )(vals, idx)
    return topk_id


# speedup vs baseline: 2.1374x; 2.1374x over previous
import jax
import jax.numpy as jnp
from jax import lax
from jax.experimental import pallas as pl
from jax.experimental.pallas import tpu as pltpu

D = 1024
V = 100000
RB = 8
NSTEP = D // RB
NBUF = 8


def _body(x_ref, w_hbm, b_ref, out_ref, acc, *bufs_and_sems):
    bufs = bufs_and_sems[:NBUF]
    sems = bufs_and_sems[NBUF:]

    def start(g, bi):
        pltpu.make_async_copy(
            w_hbm.at[pl.ds(g * RB, RB), :], bufs[bi], sems[bi]).start()

    def wait(g, bi):
        pltpu.make_async_copy(
            w_hbm.at[pl.ds(g * RB, RB), :], bufs[bi], sems[bi]).wait()

    for bi in range(NBUF):
        start(bi, bi)

    acc[...] = jnp.zeros_like(acc)

    def tbody(t, _):
        for bi in range(NBUF):
            g = t * NBUF + bi
            wait(g, bi)
            acc[...] += bufs[bi][...] * x_ref[pl.ds(g * RB, RB), :]

            @pl.when(g + NBUF < NSTEP)
            def _():
                start(g + NBUF, bi)
        return 0

    lax.fori_loop(0, NSTEP // NBUF, tbody, 0, unroll=False)

    logits = jnp.sum(acc[...], axis=0, keepdims=True) + b_ref[...]
    col = lax.broadcasted_iota(jnp.int32, (1, V), 1)
    m = jnp.max(logits)
    out_ref[0] = jnp.min(jnp.where(logits == m, col, V))


def kernel(x, W, b):
    x0 = x[0].reshape(D, 1)
    b2 = b.reshape(1, V)
    topk_id = pl.pallas_call(
        _body,
        in_specs=[
            pl.BlockSpec(memory_space=pltpu.VMEM),
            pl.BlockSpec(memory_space=pltpu.HBM),
            pl.BlockSpec(memory_space=pltpu.VMEM),
        ],
        out_specs=pl.BlockSpec(memory_space=pltpu.SMEM),
        out_shape=jax.ShapeDtypeStruct((1,), jnp.int32),
        scratch_shapes=(
            [pltpu.VMEM((RB, V), jnp.float32)]
            + [pltpu.VMEM((RB, V), jnp.float32) for _ in range(NBUF)]
            + [pltpu.SemaphoreType.DMA for _ in range(NBUF)]
        ),
    )(x0, W, b2)
    return topk_id


# speedup vs baseline: 2.1431x; 1.0027x over previous
import jax
import jax.numpy as jnp
from jax import lax
from jax.experimental import pallas as pl
from jax.experimental.pallas import tpu as pltpu

D = 1024
V = 100000
RB = 8
NSTEP = D // RB
NBUF = 8


def _body(x_ref, w_hbm, b_ref, out_ref, acc, *bufs_and_sems):
    bufs = bufs_and_sems[:NBUF]
    sems = bufs_and_sems[NBUF:]

    def start(g, bi):
        pltpu.async_copy(
            w_hbm.at[pl.ds(g * RB, RB), :], bufs[bi], sems[bi],
            priority=bi % 2)

    def wait(g, bi):
        pltpu.make_async_copy(
            w_hbm.at[pl.ds(g * RB, RB), :], bufs[bi], sems[bi]).wait()

    for bi in range(NBUF):
        start(bi, bi)

    acc[...] = jnp.zeros_like(acc)

    def tbody(t, _):
        for bi in range(NBUF):
            g = t * NBUF + bi
            wait(g, bi)
            acc[...] += bufs[bi][...] * x_ref[pl.ds(g * RB, RB), :]

            @pl.when(g + NBUF < NSTEP)
            def _():
                start(g + NBUF, bi)
        return 0

    lax.fori_loop(0, NSTEP // NBUF, tbody, 0, unroll=False)

    logits = jnp.sum(acc[...], axis=0, keepdims=True) + b_ref[...]
    col = lax.broadcasted_iota(jnp.int32, (1, V), 1)
    m = jnp.max(logits)
    out_ref[0] = jnp.min(jnp.where(logits == m, col, V))


def kernel(x, W, b):
    x0 = x[0].reshape(D, 1)
    b2 = b.reshape(1, V)
    topk_id = pl.pallas_call(
        _body,
        in_specs=[
            pl.BlockSpec(memory_space=pltpu.VMEM),
            pl.BlockSpec(memory_space=pltpu.HBM),
            pl.BlockSpec(memory_space=pltpu.VMEM),
        ],
        out_specs=pl.BlockSpec(memory_space=pltpu.SMEM),
        out_shape=jax.ShapeDtypeStruct((1,), jnp.int32),
        scratch_shapes=(
            [pltpu.VMEM((RB, V), jnp.float32)]
            + [pltpu.VMEM((RB, V), jnp.float32) for _ in range(NBUF)]
            + [pltpu.SemaphoreType.DMA for _ in range(NBUF)]
        ),
    )(x0, W, b2)
    return topk_id
